# baseline (speedup 1.0000x reference)
import jax
import jax.numpy as jnp
from jax.experimental import pallas as pl
from jax.experimental.pallas import tpu as pltpu

N, C, T = 64, 256, 512
K = 8
ALPHA = 0.8


def _dyn_graph_body(gamma_ref, x_ref, ap_ref, xo_ref, ao_ref):
    xv = x_ref[0]
    ap = ap_ref[0]
    gamma = gamma_ref[0]

    mean = jnp.mean(xv, axis=1, keepdims=True)
    xm = xv - mean
    var = jnp.sum(xm * xm, axis=1, keepdims=True) * (1.0 / (T - 1))
    std = jnp.sqrt(var) + 1e-06
    xn = xm / std

    acc = jax.lax.dot_general(
        xn, xn, (((1,), (1,)), ((), ())),
        preferred_element_type=jnp.float32)
    A = jnp.maximum(acc * (1.0 / T), 0.0)

    col = jax.lax.broadcasted_iota(jnp.int32, (C, C), 1)
    work = A
    mask = jnp.zeros((C, C), dtype=jnp.float32)
    for _ in range(K):
        m = jnp.max(work, axis=1, keepdims=True)
        cand = jnp.where(work >= m, col, C)
        first = jnp.min(cand, axis=1, keepdims=True)
        sel = col == first
        mask = jnp.where(sel, 1.0, mask)
        work = jnp.where(sel, -1.0, work)
    A = A * mask

    A = 0.5 * (A + A.T)
    row = jax.lax.broadcasted_iota(jnp.int32, (C, C), 0)
    A = jnp.where(row == col, A + 1.0, A)
    deg = jnp.sum(A, axis=1, keepdims=True) + 1e-06
    A = A / deg

    A = ALPHA * ap + (1.0 - ALPHA) * A
    ao_ref[0] = A

    z = jax.lax.dot_general(
        A, xv, (((1,), (0,)), ((), ())),
        preferred_element_type=jnp.float32)
    xo_ref[0] = xv + gamma * z


def kernel(x, A_prev, gamma):
    gamma_arr = jnp.reshape(gamma.astype(jnp.float32), (1,))
    grid_spec = pltpu.PrefetchScalarGridSpec(
        num_scalar_prefetch=1,
        grid=(N,),
        in_specs=[
            pl.BlockSpec((1, C, T), lambda i, g: (i, 0, 0)),
            pl.BlockSpec((1, C, C), lambda i, g: (i, 0, 0)),
        ],
        out_specs=[
            pl.BlockSpec((1, C, T), lambda i, g: (i, 0, 0)),
            pl.BlockSpec((1, C, C), lambda i, g: (i, 0, 0)),
        ],
    )
    x_out, A_out = pl.pallas_call(
        _dyn_graph_body,
        grid_spec=grid_spec,
        out_shape=[
            jax.ShapeDtypeStruct((N, C, T), jnp.float32),
            jax.ShapeDtypeStruct((N, C, C), jnp.float32),
        ],
        compiler_params=pltpu.CompilerParams(
            dimension_semantics=("parallel",),
        ),
    )(gamma_arr, x, A_prev)
    return (x_out, A_out)


# speedup vs baseline: 1.7249x; 1.7249x over previous
import jax
import jax.numpy as jnp
from jax.experimental import pallas as pl
from jax.experimental.pallas import tpu as pltpu

N, C, T = 64, 256, 512
K = 8
ALPHA = 0.8


def _dyn_graph_body(gamma_ref, x_ref, ap_ref, xo_ref, ao_ref):
    xv = x_ref[0]
    ap = ap_ref[0]
    gamma = gamma_ref[0]

    mean = jnp.mean(xv, axis=1, keepdims=True)
    xm = xv - mean
    var = jnp.sum(xm * xm, axis=1, keepdims=True) * (1.0 / (T - 1))
    std = jnp.sqrt(var) + 1e-06
    xn = xm / std

    acc = jax.lax.dot_general(
        xn, xn, (((1,), (1,)), ((), ())),
        preferred_element_type=jnp.float32)
    A = jnp.maximum(acc * (1.0 / T), 0.0)

    work = A
    for _ in range(K - 1):
        m = jnp.max(work, axis=1, keepdims=True)
        work = jnp.where(work >= m, -1.0, work)
    thr = jnp.max(work, axis=1, keepdims=True)
    A = jnp.where(A >= thr, A, 0.0)

    A = 0.5 * (A + A.T)
    row = jax.lax.broadcasted_iota(jnp.int32, (C, C), 0)
    col = jax.lax.broadcasted_iota(jnp.int32, (C, C), 1)
    A = jnp.where(row == col, A + 1.0, A)
    deg = jnp.sum(A, axis=1, keepdims=True) + 1e-06
    A = A / deg

    A = ALPHA * ap + (1.0 - ALPHA) * A
    ao_ref[0] = A

    z = jax.lax.dot_general(
        A, xv, (((1,), (0,)), ((), ())),
        preferred_element_type=jnp.float32)
    xo_ref[0] = xv + gamma * z


def kernel(x, A_prev, gamma):
    gamma_arr = jnp.reshape(gamma.astype(jnp.float32), (1,))
    grid_spec = pltpu.PrefetchScalarGridSpec(
        num_scalar_prefetch=1,
        grid=(N,),
        in_specs=[
            pl.BlockSpec((1, C, T), lambda i, g: (i, 0, 0)),
            pl.BlockSpec((1, C, C), lambda i, g: (i, 0, 0)),
        ],
        out_specs=[
            pl.BlockSpec((1, C, T), lambda i, g: (i, 0, 0)),
            pl.BlockSpec((1, C, C), lambda i, g: (i, 0, 0)),
        ],
    )
    x_out, A_out = pl.pallas_call(
        _dyn_graph_body,
        grid_spec=grid_spec,
        out_shape=[
            jax.ShapeDtypeStruct((N, C, T), jnp.float32),
            jax.ShapeDtypeStruct((N, C, C), jnp.float32),
        ],
        compiler_params=pltpu.CompilerParams(
            dimension_semantics=("parallel",),
        ),
    )(gamma_arr, x, A_prev)
    return (x_out, A_out)


# speedup vs baseline: 2.8304x; 1.6410x over previous
import jax
import jax.numpy as jnp
from jax.experimental import pallas as pl
from jax.experimental.pallas import tpu as pltpu

N, C, T = 64, 256, 512
K = 8
ALPHA = 0.8
B = 2


def _dyn_graph_body(gamma_ref, x_ref, ap_ref, xo_ref, ao_ref):
    xv = x_ref[...]
    ap = ap_ref[...]
    gamma = gamma_ref[0]

    mean = jnp.mean(xv, axis=2, keepdims=True)
    xm = xv - mean
    var = jnp.sum(xm * xm, axis=2, keepdims=True) * (1.0 / (T - 1))
    sinv = 1.0 / (jnp.sqrt(var) + 1e-06)

    acc = jax.lax.dot_general(
        xm, xm, (((2,), (2,)), ((0,), (0,))),
        preferred_element_type=jnp.float32)
    scale = (sinv * (1.0 / T)) * jnp.swapaxes(sinv, 1, 2)
    A = jnp.maximum(acc * scale, 0.0)

    work = A
    for _ in range(K - 1):
        m = jnp.max(work, axis=2, keepdims=True)
        work = jnp.where(work >= m, -1.0, work)
    thr = jnp.max(work, axis=2, keepdims=True)

    mrow = jnp.where(A >= thr, A, 0.0)
    mcol = jnp.where(A >= jnp.swapaxes(thr, 1, 2), A, 0.0)
    S = 0.5 * (mrow + mcol)

    row = jax.lax.broadcasted_iota(jnp.int32, (B, C, C), 1)
    col = jax.lax.broadcasted_iota(jnp.int32, (B, C, C), 2)
    S = jnp.where(row == col, S + 1.0, S)
    deg = jnp.sum(S, axis=2, keepdims=True) + 1e-06
    S = S / deg
    A_out = ALPHA * ap + (1.0 - ALPHA) * S
    ao_ref[...] = A_out

    z = jax.lax.dot_general(
        A_out, xv, (((2,), (1,)), ((0,), (0,))),
        preferred_element_type=jnp.float32)
    xo_ref[...] = xv + gamma * z


def kernel(x, A_prev, gamma):
    gamma_arr = jnp.reshape(gamma.astype(jnp.float32), (1,))
    grid_spec = pltpu.PrefetchScalarGridSpec(
        num_scalar_prefetch=1,
        grid=(N // B,),
        in_specs=[
            pl.BlockSpec((B, C, T), lambda i, g: (i, 0, 0)),
            pl.BlockSpec((B, C, C), lambda i, g: (i, 0, 0)),
        ],
        out_specs=[
            pl.BlockSpec((B, C, T), lambda i, g: (i, 0, 0)),
            pl.BlockSpec((B, C, C), lambda i, g: (i, 0, 0)),
        ],
    )
    x_out, A_out = pl.pallas_call(
        _dyn_graph_body,
        grid_spec=grid_spec,
        out_shape=[
            jax.ShapeDtypeStruct((N, C, T), jnp.float32),
            jax.ShapeDtypeStruct((N, C, C), jnp.float32),
        ],
        compiler_params=pltpu.CompilerParams(
            dimension_semantics=("parallel",),
        ),
    )(gamma_arr, x, A_prev)
    return (x_out, A_out)


# speedup vs baseline: 3.2405x; 1.1449x over previous
import jax
import jax.numpy as jnp
from jax.experimental import pallas as pl
from jax.experimental.pallas import tpu as pltpu

N, C, T = 64, 256, 512
K = 8
ALPHA = 0.8
B = 4


def _dyn_graph_body(gamma_ref, x_ref, ap_ref, xo_ref, ao_ref):
    xv = x_ref[...]
    ap = ap_ref[...]
    gamma = gamma_ref[0]

    mean = jnp.mean(xv, axis=2, keepdims=True)
    xm = xv - mean
    var = jnp.sum(xm * xm, axis=2, keepdims=True) * (1.0 / (T - 1))
    sinv = 1.0 / (jnp.sqrt(var) + 1e-06)

    acc = jax.lax.dot_general(
        xm, xm, (((2,), (2,)), ((0,), (0,))),
        preferred_element_type=jnp.float32)
    scale = (sinv * (1.0 / T)) * jnp.swapaxes(sinv, 1, 2)
    A = jnp.maximum(acc * scale, 0.0)

    work = A
    for _ in range(K - 1):
        m = jnp.max(work, axis=2, keepdims=True)
        work = jnp.where(work >= m, -1.0, work)
    thr = jnp.max(work, axis=2, keepdims=True)

    mrow = jnp.where(A >= thr, A, 0.0)
    mcol = jnp.where(A >= jnp.swapaxes(thr, 1, 2), A, 0.0)
    S = 0.5 * (mrow + mcol)

    row = jax.lax.broadcasted_iota(jnp.int32, (B, C, C), 1)
    col = jax.lax.broadcasted_iota(jnp.int32, (B, C, C), 2)
    S = jnp.where(row == col, S + 1.0, S)
    deg = jnp.sum(S, axis=2, keepdims=True) + 1e-06
    S = S / deg
    A_out = ALPHA * ap + (1.0 - ALPHA) * S
    ao_ref[...] = A_out

    z = jax.lax.dot_general(
        A_out, xv, (((2,), (1,)), ((0,), (0,))),
        preferred_element_type=jnp.float32)
    xo_ref[...] = xv + gamma * z


def kernel(x, A_prev, gamma):
    gamma_arr = jnp.reshape(gamma.astype(jnp.float32), (1,))
    grid_spec = pltpu.PrefetchScalarGridSpec(
        num_scalar_prefetch=1,
        grid=(N // B,),
        in_specs=[
            pl.BlockSpec((B, C, T), lambda i, g: (i, 0, 0)),
            pl.BlockSpec((B, C, C), lambda i, g: (i, 0, 0)),
        ],
        out_specs=[
            pl.BlockSpec((B, C, T), lambda i, g: (i, 0, 0)),
            pl.BlockSpec((B, C, C), lambda i, g: (i, 0, 0)),
        ],
    )
    x_out, A_out = pl.pallas_call(
        _dyn_graph_body,
        grid_spec=grid_spec,
        out_shape=[
            jax.ShapeDtypeStruct((N, C, T), jnp.float32),
            jax.ShapeDtypeStruct((N, C, C), jnp.float32),
        ],
        compiler_params=pltpu.CompilerParams(
            dimension_semantics=("parallel",),
        ),
    )(gamma_arr, x, A_prev)
    return (x_out, A_out)


# speedup vs baseline: 3.5383x; 1.0919x over previous
import jax
import jax.numpy as jnp
from jax.experimental import pallas as pl
from jax.experimental.pallas import tpu as pltpu

N, C, T = 64, 256, 512
K = 8
ALPHA = 0.8
B = 8


def _dyn_graph_body(gamma_ref, x_ref, ap_ref, xo_ref, ao_ref):
    xv = x_ref[...]
    ap = ap_ref[...]
    gamma = gamma_ref[0]

    mean = jnp.mean(xv, axis=2, keepdims=True)
    xm = xv - mean
    var = jnp.sum(xm * xm, axis=2, keepdims=True) * (1.0 / (T - 1))
    sinv = 1.0 / (jnp.sqrt(var) + 1e-06)

    acc = jax.lax.dot_general(
        xm, xm, (((2,), (2,)), ((0,), (0,))),
        preferred_element_type=jnp.float32)
    scale = (sinv * (1.0 / T)) * jnp.swapaxes(sinv, 1, 2)
    A = jnp.maximum(acc * scale, 0.0)

    work = A
    for _ in range(K - 1):
        m = jnp.max(work, axis=2, keepdims=True)
        work = jnp.where(work >= m, -1.0, work)
    thr = jnp.max(work, axis=2, keepdims=True)

    mrow = jnp.where(A >= thr, A, 0.0)
    mcol = jnp.where(A >= jnp.swapaxes(thr, 1, 2), A, 0.0)
    S = 0.5 * (mrow + mcol)

    row = jax.lax.broadcasted_iota(jnp.int32, (B, C, C), 1)
    col = jax.lax.broadcasted_iota(jnp.int32, (B, C, C), 2)
    S = jnp.where(row == col, S + 1.0, S)
    deg = jnp.sum(S, axis=2, keepdims=True) + 1e-06
    S = S / deg
    A_out = ALPHA * ap + (1.0 - ALPHA) * S
    ao_ref[...] = A_out

    z = jax.lax.dot_general(
        A_out, xv, (((2,), (1,)), ((0,), (0,))),
        preferred_element_type=jnp.float32)
    xo_ref[...] = xv + gamma * z


def kernel(x, A_prev, gamma):
    gamma_arr = jnp.reshape(gamma.astype(jnp.float32), (1,))
    grid_spec = pltpu.PrefetchScalarGridSpec(
        num_scalar_prefetch=1,
        grid=(N // B,),
        in_specs=[
            pl.BlockSpec((B, C, T), lambda i, g: (i, 0, 0)),
            pl.BlockSpec((B, C, C), lambda i, g: (i, 0, 0)),
        ],
        out_specs=[
            pl.BlockSpec((B, C, T), lambda i, g: (i, 0, 0)),
            pl.BlockSpec((B, C, C), lambda i, g: (i, 0, 0)),
        ],
    )
    x_out, A_out = pl.pallas_call(
        _dyn_graph_body,
        grid_spec=grid_spec,
        out_shape=[
            jax.ShapeDtypeStruct((N, C, T), jnp.float32),
            jax.ShapeDtypeStruct((N, C, C), jnp.float32),
        ],
        compiler_params=pltpu.CompilerParams(
            dimension_semantics=("parallel",),
        ),
    )(gamma_arr, x, A_prev)
    return (x_out, A_out)
